# baseline (speedup 1.0000x reference)
import functools

import jax
import jax.numpy as jnp
import numpy as np
from jax import lax
from jax.experimental import pallas as pl
from jax.experimental.pallas import tpu as pltpu
from jax.experimental.pallas import tpu_sc as plsc

_B = 16384
_DU = 64
_UV = 1000000
_SMALL_DIMS = (8, 16, 8, 8, 16, 8, 8, 8)
_SMALL_VOCAB = (4, 1000, 100, 10, 1000, 100, 10, 10)
_SP = 128
_H = 256
_DM = 128

_FLAT_OFF = tuple(int(o) for o in np.cumsum(
    [0] + [v * d for v, d in zip(_SMALL_VOCAB, _SMALL_DIMS)]))
_FLAT_LEN = _FLAT_OFF[-1]

_info = plsc.get_sparse_core_info()
_NC, _NS = _info.num_cores, _info.num_subcores
_NW = _NC * _NS
_BPW = _B // _NW
_GRP = _BPW // 16

_sc_mesh = plsc.VectorSubcoreMesh(core_axis_name="c", subcore_axis_name="s")


@functools.partial(
    pl.kernel,
    mesh=_sc_mesh,
    out_type=jax.ShapeDtypeStruct((_B, _SP), jnp.float32),
    scratch_types=(
        [pltpu.VMEM((_BPW,), jnp.int32) for _ in range(8)]
        + [
            pltpu.VMEM((_FLAT_LEN,), jnp.float32),
            pltpu.VMEM((_BPW, _SP), jnp.float32),
        ]
    ),
    compiler_params=pltpu.CompilerParams(needs_layout_passes=False),
)
def _sc_smalls(*refs):
    idx_hbm = refs[0:8]
    small_flat = refs[8]
    small_out = refs[9]
    idx_v = refs[10:18]
    tbl_v = refs[18]
    small_v = refs[19]

    wid = lax.axis_index("s") * _NC + lax.axis_index("c")
    base = wid * _BPW

    pltpu.sync_copy(small_flat, tbl_v)
    for i in range(8):
        pltpu.sync_copy(idx_hbm[i].at[pl.ds(base, _BPW)], idx_v[i])

    lane = lax.iota(jnp.int32, 16)

    def body(g, carry):
        rows = g * 16 + lane
        for t in range(8):
            d = _SMALL_DIMS[t]
            fo = _FLAT_OFF[t]
            idx16 = idx_v[t][pl.ds(g * 16, 16)]
            addr = idx16 * d + fo
            for j in range(d):
                vals = plsc.load_gather(tbl_v, [addr + j])
                plsc.store_scatter(
                    small_v,
                    [rows, jnp.full((16,), 16 * t + j, jnp.int32)],
                    vals,
                )
        return carry

    lax.fori_loop(0, _GRP, body, 0)

    pltpu.sync_copy(small_v, small_out.at[pl.ds(base, _BPW)])


@functools.partial(
    pl.kernel,
    mesh=_sc_mesh,
    out_type=jax.ShapeDtypeStruct((_B, _DU), jnp.float32),
    scratch_types=(
        pltpu.VMEM((_BPW,), jnp.int32),
        pltpu.SemaphoreType.DMA,
    ),
    compiler_params=pltpu.CompilerParams(needs_layout_passes=False),
)
def _sc_user(idx_hbm, user_tbl, user_out, uidx_v, usem):
    wid = lax.axis_index("s") * _NC + lax.axis_index("c")
    base = wid * _BPW

    pltpu.sync_copy(idx_hbm.at[pl.ds(base, _BPW)], uidx_v)

    def body(g, carry):
        uidx16 = uidx_v[pl.ds(g * 16, 16)]
        for j in range(16):
            s = uidx16[j]
            pltpu.async_copy(
                user_tbl.at[pl.ds(s, 1)],
                user_out.at[pl.ds(base + g * 16 + j, 1)],
                usem,
            )
        return carry

    lax.fori_loop(0, _GRP, body, 0)

    pltpu.make_async_copy(
        user_tbl.at[pl.ds(0, _BPW)],
        user_out.at[pl.ds(base, _BPW)],
        usem,
    ).wait()


_BM = 1024


def _ffn_body(xu_ref, xs_ref, w1u_ref, w1s_ref, b1_ref, w2_ref, b2_ref,
              o_ref):
    acc = (
        jnp.dot(xu_ref[...], w1u_ref[...], preferred_element_type=jnp.float32)
        + jnp.dot(xs_ref[...], w1s_ref[...],
                  preferred_element_type=jnp.float32)
        + b1_ref[...]
    )
    h = jnp.maximum(acc, 0.0)
    o_ref[...] = (
        jnp.dot(h, w2_ref[...], preferred_element_type=jnp.float32)
        + b2_ref[...]
    )


_ffn = pl.pallas_call(
    _ffn_body,
    grid=(_B // _BM,),
    in_specs=[
        pl.BlockSpec((_BM, _DU), lambda i: (i, 0)),
        pl.BlockSpec((_BM, _SP), lambda i: (i, 0)),
        pl.BlockSpec((_DU, _H), lambda i: (0, 0)),
        pl.BlockSpec((_SP, _H), lambda i: (0, 0)),
        pl.BlockSpec((1, _H), lambda i: (0, 0)),
        pl.BlockSpec((_H, _DM), lambda i: (0, 0)),
        pl.BlockSpec((1, _DM), lambda i: (0, 0)),
    ],
    out_specs=pl.BlockSpec((_BM, _DM), lambda i: (i, 0)),
    out_shape=jax.ShapeDtypeStruct((_B, _DM), jnp.float32),
)

_PACK_ROWS = np.concatenate(
    [16 * t + np.arange(d) for t, d in enumerate(_SMALL_DIMS)])


def kernel(useruin, gender, region_code, language, platform, device, age,
           grade, city_level, user_table, gender_table, region_table,
           language_table, platform_table, device_table, age_table,
           grade_table, city_level_table, W1, b1, W2, b2):
    idxs = [
        x.astype(jnp.int32)
        for x in (useruin, gender, region_code, language, platform, device,
                  age, grade, city_level)
    ]
    small_flat = jnp.concatenate([
        t.reshape(-1)
        for t in (gender_table, region_table, language_table, platform_table,
                  device_table, age_table, grade_table, city_level_table)
    ])
    small_emb = _sc_smalls(*idxs[1:], small_flat)
    user_emb = _sc_user(idxs[0], user_table)

    w1s = jnp.zeros((_SP, _H), jnp.float32).at[_PACK_ROWS].set(W1[_DU:])
    return _ffn(user_emb, small_emb, W1[:_DU], w1s,
                b1.reshape(1, _H), W2, b2.reshape(1, _DM))


# speedup vs baseline: 1.0032x; 1.0032x over previous
import functools

import jax
import jax.numpy as jnp
import numpy as np
from jax import lax
from jax.experimental import pallas as pl
from jax.experimental.pallas import tpu as pltpu
from jax.experimental.pallas import tpu_sc as plsc

_B = 16384
_DU = 64
_UV = 1000000
_SMALL_DIMS = (8, 16, 8, 8, 16, 8, 8, 8)
_SMALL_VOCAB = (4, 1000, 100, 10, 1000, 100, 10, 10)
_SP = 128
_H = 256
_DM = 128

_FLAT_OFF = tuple(int(o) for o in np.cumsum(
    [0] + [v * d for v, d in zip(_SMALL_VOCAB, _SMALL_DIMS)]))
_FLAT_LEN = _FLAT_OFF[-1]

_info = plsc.get_sparse_core_info()
_NC, _NS = _info.num_cores, _info.num_subcores
_NW = _NC * _NS
_BPW = _B // _NW
_GRP = _BPW // 16

_sc_mesh = plsc.VectorSubcoreMesh(core_axis_name="c", subcore_axis_name="s")


@functools.partial(
    pl.kernel,
    mesh=_sc_mesh,
    out_type=jax.ShapeDtypeStruct((_B, _SP), jnp.float32),
    scratch_types=(
        [pltpu.VMEM((_BPW,), jnp.int32) for _ in range(8)]
        + [
            pltpu.VMEM((_FLAT_LEN,), jnp.float32),
            pltpu.VMEM((_BPW, _SP), jnp.float32),
        ]
    ),
    compiler_params=pltpu.CompilerParams(needs_layout_passes=False),
)
def _sc_smalls(*refs):
    idx_hbm = refs[0:8]
    small_flat = refs[8]
    small_out = refs[9]
    idx_v = refs[10:18]
    tbl_v = refs[18]
    small_v = refs[19]

    wid = lax.axis_index("s") * _NC + lax.axis_index("c")
    base = wid * _BPW

    pltpu.sync_copy(small_flat, tbl_v)
    for i in range(8):
        pltpu.sync_copy(idx_hbm[i].at[pl.ds(base, _BPW)], idx_v[i])

    lane = lax.iota(jnp.int32, 16)

    def body(g, carry):
        rows = g * 16 + lane
        for t in range(8):
            d = _SMALL_DIMS[t]
            fo = _FLAT_OFF[t]
            idx16 = idx_v[t][pl.ds(g * 16, 16)]
            addr = idx16 * d + fo
            for j in range(d):
                vals = plsc.load_gather(tbl_v, [addr + j])
                plsc.store_scatter(
                    small_v,
                    [rows, jnp.full((16,), 16 * t + j, jnp.int32)],
                    vals,
                )
        return carry

    lax.fori_loop(0, _GRP, body, 0)

    pltpu.sync_copy(small_v, small_out.at[pl.ds(base, _BPW)])


@functools.partial(
    pl.kernel,
    mesh=_sc_mesh,
    out_type=jax.ShapeDtypeStruct((_B, _DU), jnp.float32),
    scratch_types=(
        pltpu.VMEM((_BPW,), jnp.int32),
        pltpu.SemaphoreType.DMA,
    ),
    compiler_params=pltpu.CompilerParams(needs_layout_passes=False),
)
def _sc_user(idx_hbm, user_tbl, user_out, uidx_v, usem):
    wid = lax.axis_index("s") * _NC + lax.axis_index("c")
    base = wid * _BPW

    pltpu.sync_copy(idx_hbm.at[pl.ds(base, _BPW)], uidx_v)

    @plsc.parallel_loop(0, _GRP)
    def body(g):
        uidx16 = uidx_v[pl.ds(g * 16, 16)]
        for j in range(16):
            s = uidx16[j]
            pltpu.async_copy(
                user_tbl.at[pl.ds(s, 1)],
                user_out.at[pl.ds(base + g * 16 + j, 1)],
                usem,
            )

    pltpu.make_async_copy(
        user_tbl.at[pl.ds(0, _BPW)],
        user_out.at[pl.ds(base, _BPW)],
        usem,
    ).wait()


_BM = 1024


def _ffn_body(xu_ref, xs_ref, w1u_ref, w1s_ref, b1_ref, w2_ref, b2_ref,
              o_ref):
    acc = (
        jnp.dot(xu_ref[...], w1u_ref[...], preferred_element_type=jnp.float32)
        + jnp.dot(xs_ref[...], w1s_ref[...],
                  preferred_element_type=jnp.float32)
        + b1_ref[...]
    )
    h = jnp.maximum(acc, 0.0)
    o_ref[...] = (
        jnp.dot(h, w2_ref[...], preferred_element_type=jnp.float32)
        + b2_ref[...]
    )


_ffn = pl.pallas_call(
    _ffn_body,
    grid=(_B // _BM,),
    in_specs=[
        pl.BlockSpec((_BM, _DU), lambda i: (i, 0)),
        pl.BlockSpec((_BM, _SP), lambda i: (i, 0)),
        pl.BlockSpec((_DU, _H), lambda i: (0, 0)),
        pl.BlockSpec((_SP, _H), lambda i: (0, 0)),
        pl.BlockSpec((1, _H), lambda i: (0, 0)),
        pl.BlockSpec((_H, _DM), lambda i: (0, 0)),
        pl.BlockSpec((1, _DM), lambda i: (0, 0)),
    ],
    out_specs=pl.BlockSpec((_BM, _DM), lambda i: (i, 0)),
    out_shape=jax.ShapeDtypeStruct((_B, _DM), jnp.float32),
)

_PACK_ROWS = np.concatenate(
    [16 * t + np.arange(d) for t, d in enumerate(_SMALL_DIMS)])


def kernel(useruin, gender, region_code, language, platform, device, age,
           grade, city_level, user_table, gender_table, region_table,
           language_table, platform_table, device_table, age_table,
           grade_table, city_level_table, W1, b1, W2, b2):
    idxs = [
        x.astype(jnp.int32)
        for x in (useruin, gender, region_code, language, platform, device,
                  age, grade, city_level)
    ]
    small_flat = jnp.concatenate([
        t.reshape(-1)
        for t in (gender_table, region_table, language_table, platform_table,
                  device_table, age_table, grade_table, city_level_table)
    ])
    small_emb = _sc_smalls(*idxs[1:], small_flat)
    user_emb = _sc_user(idxs[0], user_table)

    w1s = jnp.zeros((_SP, _H), jnp.float32).at[_PACK_ROWS].set(W1[_DU:])
    return _ffn(user_emb, small_emb, W1[:_DU], w1s,
                b1.reshape(1, _H), W2, b2.reshape(1, _DM))
